# baseline (speedup 1.0000x reference)
import functools

import jax
import jax.numpy as jnp
from jax import lax
from jax.experimental import pallas as pl
from jax.experimental.pallas import tpu as pltpu
from jax.experimental.pallas import tpu_sc as plsc

N_FIELDS = 26
VOCAB = 100000
DIM = 64
BATCH = 4096

NW = 32
NBLK = N_FIELDS * 8
BPW = (NBLK + NW - 1) // NW
CHUNK_V = 2048
NBUF = 4
VMAIN = (VOCAB // 128) * 128
VTAIL = VOCAB - VMAIN
NCH = (VMAIN + CHUNK_V - 1) // CHUNK_V
NVREG = BATCH // 16
PHI_PAD = ((NCH + 15) // 16) * 16


@functools.partial(
    pl.kernel,
    mesh=plsc.VectorSubcoreMesh(core_axis_name="c", subcore_axis_name="s"),
    out_type=jax.ShapeDtypeStruct((N_FIELDS, DIM, BATCH), jnp.float32),
    compiler_params=pltpu.CompilerParams(needs_layout_passes=False),
    scratch_types=[
        pltpu.VMEM((BATCH // 128, 128), jnp.int32),
        pltpu.VMEM((BATCH // 128, 128), jnp.int32),
        pltpu.VMEM((1, PHI_PAD), jnp.int32),
        pltpu.VMEM((NBUF, 8, CHUNK_V), jnp.float32),
        pltpu.VMEM((8, VTAIL), jnp.float32),
        pltpu.VMEM((8, BATCH), jnp.float32),
        pltpu.SemaphoreType.DMA,
        pltpu.SemaphoreType.DMA,
        pltpu.SemaphoreType.DMA,
        pltpu.SemaphoreType.DMA,
    ],
)
def _emb_sweep(sv_hbm, pm_hbm, phi_hbm, tbl_hbm, tail_hbm, out_hbm,
               sv_v, pm_v, phi_v, slab, tailb, outb, sem0, sem1, sem2, sem3):
    sems = (sem0, sem1, sem2, sem3)
    wid = lax.axis_index("s") * 2 + lax.axis_index("c")
    lane = lax.iota(jnp.int32, 16)

    def do_block(blk):
        i = blk // 8
        db8 = pl.multiple_of((blk % 8) * 8, 8)
        pltpu.sync_copy(sv_hbm.at[i], sv_v)
        pltpu.sync_copy(pm_hbm.at[i], pm_v)
        pltpu.sync_copy(phi_hbm.at[i], phi_v)
        pltpu.sync_copy(tail_hbm.at[i, pl.ds(db8, 8), :], tailb)
        phi_vecs = [phi_v[0, pl.ds(16 * k, 16)] for k in range(PHI_PAD // 16)]

        def phi_at(c):
            return jnp.sum(
                jnp.where(lane == (c % 16), phi_vecs[c // 16], 0))

        def chunk_len(c):
            return min(CHUNK_V, VMAIN - c * CHUNK_V)

        def start_chunk(c):
            clen = chunk_len(c)
            src = tbl_hbm.at[i, pl.ds(db8, 8), pl.ds(c * CHUNK_V, clen)]
            if clen == CHUNK_V:
                dst = slab.at[c % NBUF]
            else:
                dst = slab.at[c % NBUF, :, pl.ds(0, clen)]
            return pltpu.async_copy(src, dst, sems[c % NBUF])

        def process_vreg(j, c, clen):
            row = j // 8
            col = (j % 8) * 16
            v = sv_v[row, pl.ds(col, 16)]
            b = pm_v[row, pl.ds(col, 16)]
            lv = v - c * CHUNK_V
            mask = (lv >= 0) & (lv < clen)
            lvc = jnp.where(mask, lv, 0)
            for dr in range(8):
                drv = jnp.full((16,), dr, jnp.int32)
                val = plsc.load_gather(slab.at[c % NBUF], [drv, lvc],
                                       mask=mask)
                plsc.store_scatter(outb, [drv, b], val, mask=mask)
            return jnp.int32(0)

        cps = [start_chunk(c) for c in range(NBUF - 1)]
        for c in range(NCH):
            clen = chunk_len(c)
            if c + NBUF - 1 < NCH:
                cps.append(start_chunk(c + NBUF - 1))
            cps.pop(0).wait()
            jlo = jnp.int32(0) if c == 0 else phi_at(c - 1) >> 4
            jhi = (phi_at(c) + 15) >> 4
            lax.fori_loop(jlo, jhi,
                          lambda j, carry: process_vreg(j, c, clen),
                          jnp.int32(0))

        def tail_body(j, carry):
            row = j // 8
            col = (j % 8) * 16
            v = sv_v[row, pl.ds(col, 16)]
            b = pm_v[row, pl.ds(col, 16)]
            lv = v - VMAIN
            mask = lv >= 0
            lvc = jnp.where(mask, lv, 0)
            for dr in range(8):
                drv = jnp.full((16,), dr, jnp.int32)
                val = plsc.load_gather(tailb, [drv, lvc], mask=mask)
                plsc.store_scatter(outb, [drv, b], val, mask=mask)
            return carry

        lax.fori_loop(phi_at(NCH - 1) >> 4, jnp.int32(NVREG), tail_body,
                      jnp.int32(0))
        pltpu.sync_copy(outb, out_hbm.at[i, pl.ds(db8, 8), :])

    def round_body(k, carry):
        blk = k * NW + wid

        @pl.when(blk < NBLK)
        def _():
            do_block(blk)

        return carry

    lax.fori_loop(0, BPW, round_body, jnp.int32(0))


def kernel(X, tables):
    xt = X.astype(jnp.int32).T
    perm = jnp.argsort(xt, axis=1).astype(jnp.int32)
    sv = jnp.take_along_axis(xt, perm, axis=1)
    sv3 = sv.reshape(N_FIELDS, BATCH // 128, 128)
    pm3 = perm.reshape(N_FIELDS, BATCH // 128, 128)
    edges = jnp.array(
        [min((c + 1) * CHUNK_V, VMAIN) for c in range(NCH)], jnp.int32)
    phi = jax.vmap(lambda row: jnp.searchsorted(row, edges))(sv)
    phi = jnp.pad(phi.astype(jnp.int32), ((0, 0), (0, PHI_PAD - NCH)))
    phi3 = phi.reshape(N_FIELDS, 1, PHI_PAD)
    tbl_t = tables.transpose(0, 2, 1)
    tail_t = tbl_t[:, :, VMAIN:]
    out_t = _emb_sweep(sv3, pm3, phi3, tbl_t, tail_t)
    return out_t.transpose(0, 2, 1).reshape(N_FIELDS, BATCH, 1, DIM)


# speedup vs baseline: 1.0683x; 1.0683x over previous
import functools

import jax
import jax.numpy as jnp
from jax import lax
from jax.experimental import pallas as pl
from jax.experimental.pallas import tpu as pltpu
from jax.experimental.pallas import tpu_sc as plsc

N_FIELDS = 26
VOCAB = 100000
DIM = 64
BATCH = 4096

NW = 32
NBLK = N_FIELDS * 8
BPW = (NBLK + NW - 1) // NW
CHUNK_V = 4096
NBUF = 2
VMAIN = (VOCAB // 128) * 128
VTAIL = VOCAB - VMAIN
NCH = (VMAIN + CHUNK_V - 1) // CHUNK_V
NVREG = BATCH // 16
PHI_PAD = ((NCH + 15) // 16) * 16


@functools.partial(
    pl.kernel,
    mesh=plsc.VectorSubcoreMesh(core_axis_name="c", subcore_axis_name="s"),
    out_type=jax.ShapeDtypeStruct((N_FIELDS, DIM, BATCH), jnp.float32),
    compiler_params=pltpu.CompilerParams(needs_layout_passes=False),
    scratch_types=[
        pltpu.VMEM((BATCH // 128, 128), jnp.int32),
        pltpu.VMEM((BATCH // 128, 128), jnp.int32),
        pltpu.VMEM((1, PHI_PAD), jnp.int32),
        pltpu.VMEM((NBUF, 8, CHUNK_V), jnp.float32),
        pltpu.VMEM((8, VTAIL), jnp.float32),
        pltpu.VMEM((8, BATCH), jnp.float32),
        pltpu.SemaphoreType.DMA,
        pltpu.SemaphoreType.DMA,
    ],
)
def _emb_sweep(sv_hbm, pm_hbm, phi_hbm, tbl_hbm, tail_hbm, out_hbm,
               sv_v, pm_v, phi_v, slab, tailb, outb, sem0, sem1):
    sems = (sem0, sem1)
    wid = lax.axis_index("s") * 2 + lax.axis_index("c")
    lane = lax.iota(jnp.int32, 16)

    def do_block(blk):
        i = blk // 8
        db8 = pl.multiple_of((blk % 8) * 8, 8)
        pltpu.sync_copy(sv_hbm.at[i], sv_v)
        pltpu.sync_copy(pm_hbm.at[i], pm_v)
        pltpu.sync_copy(phi_hbm.at[i], phi_v)
        pltpu.sync_copy(tail_hbm.at[i, pl.ds(db8, 8), :], tailb)
        phi_vecs = [phi_v[0, pl.ds(16 * k, 16)] for k in range(PHI_PAD // 16)]

        def phi_at(c):
            return jnp.sum(
                jnp.where(lane == (c % 16), phi_vecs[c // 16], 0))

        def chunk_len(c):
            return min(CHUNK_V, VMAIN - c * CHUNK_V)

        def start_chunk(c):
            clen = chunk_len(c)
            src = tbl_hbm.at[i, pl.ds(db8, 8), pl.ds(c * CHUNK_V, clen)]
            if clen == CHUNK_V:
                dst = slab.at[c % NBUF]
            else:
                dst = slab.at[c % NBUF, :, pl.ds(0, clen)]
            return pltpu.async_copy(src, dst, sems[c % NBUF])

        def process_vreg(j, c, clen):
            row = j // 8
            col = (j % 8) * 16
            v = sv_v[row, pl.ds(col, 16)]
            b = pm_v[row, pl.ds(col, 16)]
            lv = v - c * CHUNK_V
            mask = (lv >= 0) & (lv < clen)
            lvc = jnp.where(mask, lv, 0)
            for dr in range(8):
                drv = jnp.full((16,), dr, jnp.int32)
                val = plsc.load_gather(slab.at[c % NBUF], [drv, lvc],
                                       mask=mask)
                plsc.store_scatter(outb, [drv, b], val, mask=mask)
            return jnp.int32(0)

        cps = [start_chunk(c) for c in range(NBUF - 1)]
        for c in range(NCH):
            clen = chunk_len(c)
            if c + NBUF - 1 < NCH:
                cps.append(start_chunk(c + NBUF - 1))
            cps.pop(0).wait()
            jlo = jnp.int32(0) if c == 0 else phi_at(c - 1) >> 4
            jhi = (phi_at(c) + 15) >> 4
            lax.fori_loop(jlo, jhi,
                          lambda j, carry: process_vreg(j, c, clen),
                          jnp.int32(0))

        def tail_body(j, carry):
            row = j // 8
            col = (j % 8) * 16
            v = sv_v[row, pl.ds(col, 16)]
            b = pm_v[row, pl.ds(col, 16)]
            lv = v - VMAIN
            mask = lv >= 0
            lvc = jnp.where(mask, lv, 0)
            for dr in range(8):
                drv = jnp.full((16,), dr, jnp.int32)
                val = plsc.load_gather(tailb, [drv, lvc], mask=mask)
                plsc.store_scatter(outb, [drv, b], val, mask=mask)
            return carry

        lax.fori_loop(phi_at(NCH - 1) >> 4, jnp.int32(NVREG), tail_body,
                      jnp.int32(0))
        pltpu.sync_copy(outb, out_hbm.at[i, pl.ds(db8, 8), :])

    def round_body(k, carry):
        blk = k * NW + wid

        @pl.when(blk < NBLK)
        def _():
            do_block(blk)

        return carry

    lax.fori_loop(0, BPW, round_body, jnp.int32(0))


def kernel(X, tables):
    xt = X.astype(jnp.int32).T
    iota_b = jnp.broadcast_to(
        jnp.arange(BATCH, dtype=jnp.int32), (N_FIELDS, BATCH))
    sv, perm = lax.sort((xt, iota_b), dimension=1, num_keys=1)
    sv3 = sv.reshape(N_FIELDS, BATCH // 128, 128)
    pm3 = perm.reshape(N_FIELDS, BATCH // 128, 128)
    edges = jnp.array(
        [min((c + 1) * CHUNK_V, VMAIN) for c in range(NCH)], jnp.int32)
    phi = jax.vmap(lambda row: jnp.searchsorted(row, edges))(sv)
    phi = jnp.pad(phi.astype(jnp.int32), ((0, 0), (0, PHI_PAD - NCH)))
    phi3 = phi.reshape(N_FIELDS, 1, PHI_PAD)
    tbl_t = tables.transpose(0, 2, 1)
    tail_t = tbl_t[:, :, VMAIN:]
    out_t = _emb_sweep(sv3, pm3, phi3, tbl_t, tail_t)
    return out_t.transpose(0, 2, 1).reshape(N_FIELDS, BATCH, 1, DIM)


# speedup vs baseline: 1.1131x; 1.0419x over previous
import functools

import jax
import jax.numpy as jnp
from jax import lax
from jax.experimental import pallas as pl
from jax.experimental.pallas import tpu as pltpu
from jax.experimental.pallas import tpu_sc as plsc

N_FIELDS = 26
VOCAB = 100000
DIM = 64
BATCH = 4096

NW = 32
NBLK = N_FIELDS * 8
BPW = (NBLK + NW - 1) // NW
CHUNK_V = 5120
NBUF = 2
VMAIN = (VOCAB // 128) * 128
VTAIL = VOCAB - VMAIN
NCH = (VMAIN + CHUNK_V - 1) // CHUNK_V
NVREG = BATCH // 16
PHI_PAD = ((NCH + 15) // 16) * 16


@functools.partial(
    pl.kernel,
    mesh=plsc.VectorSubcoreMesh(core_axis_name="c", subcore_axis_name="s"),
    out_type=jax.ShapeDtypeStruct((N_FIELDS, DIM, BATCH), jnp.float32),
    compiler_params=pltpu.CompilerParams(needs_layout_passes=False),
    scratch_types=[
        pltpu.VMEM((BATCH // 128, 128), jnp.int32),
        pltpu.VMEM((BATCH // 128, 128), jnp.int32),
        pltpu.VMEM((1, PHI_PAD), jnp.int32),
        pltpu.VMEM((NBUF, 8, CHUNK_V), jnp.float32),
        pltpu.VMEM((8, VTAIL), jnp.float32),
        pltpu.VMEM((8, BATCH), jnp.float32),
        pltpu.SemaphoreType.DMA,
        pltpu.SemaphoreType.DMA,
    ],
)
def _emb_sweep(sv_hbm, pm_hbm, phi_hbm, tbl_hbm, tail_hbm, out_hbm,
               sv_v, pm_v, phi_v, slab, tailb, outb, sem0, sem1):
    sems = (sem0, sem1)
    wid = lax.axis_index("s") * 2 + lax.axis_index("c")
    lane = lax.iota(jnp.int32, 16)

    def do_block(blk):
        i = blk // 8
        db8 = pl.multiple_of((blk % 8) * 8, 8)
        pltpu.sync_copy(sv_hbm.at[i], sv_v)
        pltpu.sync_copy(pm_hbm.at[i], pm_v)
        pltpu.sync_copy(phi_hbm.at[i], phi_v)
        pltpu.sync_copy(tail_hbm.at[i, pl.ds(db8, 8), :], tailb)
        phi_vecs = [phi_v[0, pl.ds(16 * k, 16)] for k in range(PHI_PAD // 16)]

        def phi_at(c):
            return jnp.sum(
                jnp.where(lane == (c % 16), phi_vecs[c // 16], 0))

        def chunk_len(c):
            return min(CHUNK_V, VMAIN - c * CHUNK_V)

        def start_chunk(c):
            clen = chunk_len(c)
            src = tbl_hbm.at[i, pl.ds(db8, 8), pl.ds(c * CHUNK_V, clen)]
            if clen == CHUNK_V:
                dst = slab.at[c % NBUF]
            else:
                dst = slab.at[c % NBUF, :, pl.ds(0, clen)]
            return pltpu.async_copy(src, dst, sems[c % NBUF])

        def process_vreg(j, c, clen):
            row = j // 8
            col = (j % 8) * 16
            v = sv_v[row, pl.ds(col, 16)]
            b = pm_v[row, pl.ds(col, 16)]
            lv = v - c * CHUNK_V
            mask = (lv >= 0) & (lv < clen)
            lvc = jnp.where(mask, lv, 0)
            for dr in range(8):
                drv = jnp.full((16,), dr, jnp.int32)
                val = plsc.load_gather(slab.at[c % NBUF], [drv, lvc],
                                       mask=mask)
                plsc.store_scatter(outb, [drv, b], val, mask=mask)
            return jnp.int32(0)

        cps = [start_chunk(c) for c in range(NBUF - 1)]
        for c in range(NCH):
            clen = chunk_len(c)
            if c + NBUF - 1 < NCH:
                cps.append(start_chunk(c + NBUF - 1))
            cps.pop(0).wait()
            jlo = jnp.int32(0) if c == 0 else phi_at(c - 1) >> 4
            jhi = (phi_at(c) + 15) >> 4
            lax.fori_loop(jlo, jhi,
                          lambda j, carry: process_vreg(j, c, clen),
                          jnp.int32(0))

        def tail_body(j, carry):
            row = j // 8
            col = (j % 8) * 16
            v = sv_v[row, pl.ds(col, 16)]
            b = pm_v[row, pl.ds(col, 16)]
            lv = v - VMAIN
            mask = lv >= 0
            lvc = jnp.where(mask, lv, 0)
            for dr in range(8):
                drv = jnp.full((16,), dr, jnp.int32)
                val = plsc.load_gather(tailb, [drv, lvc], mask=mask)
                plsc.store_scatter(outb, [drv, b], val, mask=mask)
            return carry

        lax.fori_loop(phi_at(NCH - 1) >> 4, jnp.int32(NVREG), tail_body,
                      jnp.int32(0))
        pltpu.sync_copy(outb, out_hbm.at[i, pl.ds(db8, 8), :])

    def round_body(k, carry):
        blk = k * NW + wid

        @pl.when(blk < NBLK)
        def _():
            do_block(blk)

        return carry

    lax.fori_loop(0, BPW, round_body, jnp.int32(0))


def kernel(X, tables):
    xt = X.astype(jnp.int32).T
    iota_b = jnp.broadcast_to(
        jnp.arange(BATCH, dtype=jnp.int32), (N_FIELDS, BATCH))
    sv, perm = lax.sort((xt, iota_b), dimension=1, num_keys=1)
    sv3 = sv.reshape(N_FIELDS, BATCH // 128, 128)
    pm3 = perm.reshape(N_FIELDS, BATCH // 128, 128)
    edges = jnp.array(
        [min((c + 1) * CHUNK_V, VMAIN) for c in range(NCH)], jnp.int32)
    phi = jax.vmap(lambda row: jnp.searchsorted(row, edges))(sv)
    phi = jnp.pad(phi.astype(jnp.int32), ((0, 0), (0, PHI_PAD - NCH)))
    phi3 = phi.reshape(N_FIELDS, 1, PHI_PAD)
    tbl_t = tables.transpose(0, 2, 1)
    tail_t = tbl_t[:, :, VMAIN:]
    out_t = _emb_sweep(sv3, pm3, phi3, tbl_t, tail_t)
    return out_t.transpose(0, 2, 1).reshape(N_FIELDS, BATCH, 1, DIM)


# speedup vs baseline: 1.1185x; 1.0048x over previous
import functools

import jax
import jax.numpy as jnp
from jax import lax
from jax.experimental import pallas as pl
from jax.experimental.pallas import tpu as pltpu
from jax.experimental.pallas import tpu_sc as plsc

N_FIELDS = 26
VOCAB = 100000
DIM = 64
BATCH = 4096

NW = 32
NBLK = N_FIELDS * 8
BPW = (NBLK + NW - 1) // NW
CHUNK_V = 5120
NBUF = 2
VMAIN = (VOCAB // 128) * 128
VTAIL = VOCAB - VMAIN
NCH = (VMAIN + CHUNK_V - 1) // CHUNK_V
NVREG = BATCH // 16
PHI_PAD = ((NCH + 15) // 16) * 16


@functools.partial(
    pl.kernel,
    mesh=plsc.VectorSubcoreMesh(core_axis_name="c", subcore_axis_name="s"),
    out_type=jax.ShapeDtypeStruct((N_FIELDS, DIM, BATCH), jnp.float32),
    compiler_params=pltpu.CompilerParams(needs_layout_passes=False),
    scratch_types=[
        pltpu.VMEM((BATCH // 128, 128), jnp.int32),
        pltpu.VMEM((BATCH // 128, 128), jnp.int32),
        pltpu.VMEM((1, PHI_PAD), jnp.int32),
        pltpu.VMEM((NBUF, 8, CHUNK_V), jnp.float32),
        pltpu.VMEM((8, VTAIL), jnp.float32),
        pltpu.VMEM((8, BATCH), jnp.float32),
        pltpu.SemaphoreType.DMA,
        pltpu.SemaphoreType.DMA,
    ],
)
def _emb_sweep(xt_hbm, bp_hbm, phi_hbm, tbl_hbm, tail_hbm, out_hbm,
               xt_v, bp_v, phi_v, slab, tailb, outb, sem0, sem1):
    sems = (sem0, sem1)
    wid = lax.axis_index("s") * 2 + lax.axis_index("c")
    lane = lax.iota(jnp.int32, 16)

    def do_block(blk):
        i = blk // 8
        db8 = pl.multiple_of((blk % 8) * 8, 8)
        pltpu.sync_copy(xt_hbm.at[i], xt_v)
        pltpu.sync_copy(bp_hbm.at[i], bp_v)
        pltpu.sync_copy(phi_hbm.at[i], phi_v)
        pltpu.sync_copy(tail_hbm.at[i, pl.ds(db8, 8), :], tailb)
        phi_vecs = [phi_v[0, pl.ds(16 * k, 16)] for k in range(PHI_PAD // 16)]

        def phi_at(c):
            return jnp.sum(
                jnp.where(lane == (c % 16), phi_vecs[c // 16], 0))

        def chunk_len(c):
            return min(CHUNK_V, VMAIN - c * CHUNK_V)

        def start_chunk(c):
            clen = chunk_len(c)
            src = tbl_hbm.at[i, pl.ds(db8, 8), pl.ds(c * CHUNK_V, clen)]
            if clen == CHUNK_V:
                dst = slab.at[c % NBUF]
            else:
                dst = slab.at[c % NBUF, :, pl.ds(0, clen)]
            return pltpu.async_copy(src, dst, sems[c % NBUF])

        def process_vreg(j, c, clen):
            row = j // 8
            col = (j % 8) * 16
            b = bp_v[row, pl.ds(col, 16)]
            v = plsc.load_gather(xt_v, [b >> 7, b & 127])
            lv = v - c * CHUNK_V
            mask = (lv >= 0) & (lv < clen)
            lvc = jnp.where(mask, lv, 0)
            for dr in range(8):
                drv = jnp.full((16,), dr, jnp.int32)
                val = plsc.load_gather(slab.at[c % NBUF], [drv, lvc],
                                       mask=mask)
                plsc.store_scatter(outb, [drv, b], val, mask=mask)
            return jnp.int32(0)

        cps = [start_chunk(c) for c in range(NBUF - 1)]
        for c in range(NCH):
            clen = chunk_len(c)
            if c + NBUF - 1 < NCH:
                cps.append(start_chunk(c + NBUF - 1))
            cps.pop(0).wait()
            jlo = jnp.int32(0) if c == 0 else phi_at(c - 1) >> 4
            jhi = (phi_at(c) + 15) >> 4
            lax.fori_loop(jlo, jhi,
                          lambda j, carry: process_vreg(j, c, clen),
                          jnp.int32(0))

        def tail_body(j, carry):
            row = j // 8
            col = (j % 8) * 16
            b = bp_v[row, pl.ds(col, 16)]
            v = plsc.load_gather(xt_v, [b >> 7, b & 127])
            lv = v - VMAIN
            mask = lv >= 0
            lvc = jnp.where(mask, lv, 0)
            for dr in range(8):
                drv = jnp.full((16,), dr, jnp.int32)
                val = plsc.load_gather(tailb, [drv, lvc], mask=mask)
                plsc.store_scatter(outb, [drv, b], val, mask=mask)
            return carry

        lax.fori_loop(phi_at(NCH - 1) >> 4, jnp.int32(NVREG), tail_body,
                      jnp.int32(0))
        pltpu.sync_copy(outb, out_hbm.at[i, pl.ds(db8, 8), :])

    def round_body(k, carry):
        blk = k * NW + wid

        @pl.when(blk < NBLK)
        def _():
            do_block(blk)

        return carry

    lax.fori_loop(0, BPW, round_body, jnp.int32(0))


def kernel(X, tables):
    xt = X.astype(jnp.int32).T
    iota_b = jnp.broadcast_to(
        jnp.arange(BATCH, dtype=jnp.int32), (N_FIELDS, BATCH))
    ckey = jnp.where(xt >= VMAIN, NCH, xt // CHUNK_V)
    packed = ckey * BATCH + iota_b
    (sp,) = lax.sort((packed,), dimension=1)
    bp = sp % BATCH
    xt3 = xt.reshape(N_FIELDS, BATCH // 128, 128)
    bp3 = bp.reshape(N_FIELDS, BATCH // 128, 128)
    edges = jnp.arange(1, NCH + 1, dtype=jnp.int32) * BATCH
    phi = jax.vmap(lambda row: jnp.searchsorted(row, edges))(sp)
    phi = jnp.pad(phi.astype(jnp.int32), ((0, 0), (0, PHI_PAD - NCH)))
    phi3 = phi.reshape(N_FIELDS, 1, PHI_PAD)
    tbl_t = tables.transpose(0, 2, 1)
    tail_t = tbl_t[:, :, VMAIN:]
    out_t = _emb_sweep(xt3, bp3, phi3, tbl_t, tail_t)
    return out_t.transpose(0, 2, 1).reshape(N_FIELDS, BATCH, 1, DIM)
